# baseline (speedup 1.0000x reference)
import jax
import jax.numpy as jnp
from jax import lax
from jax.experimental import pallas as pl
from jax.experimental.pallas import tpu as pltpu
from jax.experimental.pallas import tpu_sc as plsc

N_NODES = 10000
D = 128
PW = D // 2
N_REL = 16
N_HOPS = 2

NC = 2
NS = 16
NW = NC * NS
C = 64
G = 8
NGRP = 20
QHALF = NGRP // 2
NCHUNK = NGRP * G
EPW_PAD = NCHUNK * C
ACC_ROWS = 10240
TRASH = N_NODES
ZPT = ACC_ROWS // NS


def _agg_body(entx_hbm, fused_hbm, heads_hbm, out_hbm,
              acc, prow0, prow1, rows0, rows1, f0, f1, h0, h1,
              gsem0, gsem1, ssem0, ssem1, isem0, isem1):
    cid = lax.axis_index("c")
    sid = lax.axis_index("s")
    wid = sid * NC + cid

    prow = (prow0, prow1)
    rows = (rows0, rows1)
    fgrp = (f0, f1)
    hgrp = (h0, h1)
    gsem = (gsem0, gsem1)
    ssem = (ssem0, ssem1)
    isem = (isem0, isem1)

    def _zrow(r, carry):
        for c8 in range(8):
            rows0[r, pl.ds(c8 * 16, 16)] = jnp.zeros((16,), jnp.float32)
        return carry
    lax.fori_loop(0, C, _zrow, 0)
    for j in range(ZPT // C):
        pltpu.sync_copy(rows0, acc.at[pl.ds(sid * ZPT + j * C, C)])

    plsc.subcore_barrier()

    def _start_idx(q, p):
        pltpu.async_copy(fused_hbm.at[wid, q], fgrp[p], isem[p])
        pltpu.async_copy(heads_hbm.at[wid, q], hgrp[p], isem[p])

    def _wait_idx(q, p):
        pltpu.make_async_copy(fused_hbm.at[wid, q], fgrp[p], isem[p]).wait()
        pltpu.make_async_copy(heads_hbm.at[wid, q], hgrp[p], isem[p]).wait()

    def _start_gather(idx_ref, b):
        pltpu.async_copy(entx_hbm.at[idx_ref], prow[b], gsem[b])

    def _wait_gather(idx_ref, b):
        pltpu.make_async_copy(entx_hbm.at[idx_ref], prow[b], gsem[b]).wait()

    def _wait_scatter(j, p, b):
        pltpu.make_async_copy(rows[b], acc.at[hgrp[p].at[j]], ssem[b]).wait()

    def _unpack(b):
        src = prow[b]
        dst = rows[b]

        def _row(r, carry):
            for k in range(PW // 16):
                v = src[r, pl.ds(16 * k, 16)]
                bc = plsc.bitcast(v, jnp.bfloat16)
                a, bb = plsc.unpack(
                    bc, format=plsc.PackFormat.INTERLEAVED,
                    preferred_element_type=jnp.float32)
                dst[r, pl.ds(32 * k, 16)] = a
                dst[r, pl.ds(32 * k + 16, 16)] = bb
            return carry
        lax.fori_loop(0, C, _row, 0)

    _start_idx(0, 0)
    _wait_idx(0, 0)
    _start_gather(fgrp[0].at[0], 0)

    def _step(t, carry):
        for p in range(2):
            q = 2 * t + p
            for j in range(G):
                b = j % 2
                if j == 2:
                    if p == 0:
                        _start_idx(q + 1, 1)
                    else:
                        @pl.when(t < QHALF - 1)
                        def _():
                            _start_idx(q + 1, 0)
                if j < G - 1:
                    _start_gather(fgrp[p].at[j + 1], 1 - b)
                elif p == 0:
                    _wait_idx(q + 1, 1)
                    _start_gather(fgrp[1].at[0], 1 - b)
                else:
                    @pl.when(t < QHALF - 1)
                    def _():
                        _wait_idx(q + 1, 0)
                        _start_gather(fgrp[0].at[0], 1 - b)
                _wait_gather(fgrp[p].at[j], b)
                if p == 0 and j < 2:
                    @pl.when(t > 0)
                    def _():
                        _wait_scatter(G - 2 + j, 1, b)
                elif j < 2:
                    _wait_scatter(G - 2 + j, 0, b)
                else:
                    _wait_scatter(j - 2, p, b)
                _unpack(b)
                pltpu.async_copy(rows[b], acc.at[hgrp[p].at[j]], ssem[b],
                                 add=True)
        return carry

    lax.fori_loop(0, QHALF, _step, 0)
    _wait_scatter(G - 2, 1, 0)
    _wait_scatter(G - 1, 1, 1)

    plsc.subcore_barrier()

    for j in range(ZPT // C):
        base = sid * ZPT + j * C
        pltpu.sync_copy(acc.at[pl.ds(base, C)], rows0)
        pltpu.sync_copy(rows0, out_hbm.at[cid, pl.ds(base, C)])


@jax.jit
def _sc_aggregate(entx, fused, heads):
    mesh = plsc.VectorSubcoreMesh(core_axis_name="c", subcore_axis_name="s")
    f = pl.kernel(
        _agg_body,
        out_type=jax.ShapeDtypeStruct((NC, ACC_ROWS, D), jnp.float32),
        mesh=mesh,
        compiler_params=pltpu.CompilerParams(
            needs_layout_passes=False, use_tc_tiling_on_sc=False
        ),
        scratch_types=[
            pltpu.VMEM_SHARED((ACC_ROWS, D), jnp.float32),
            pltpu.VMEM((C, PW), jnp.int32),
            pltpu.VMEM((C, PW), jnp.int32),
            pltpu.VMEM((C, D), jnp.float32),
            pltpu.VMEM((C, D), jnp.float32),
            pltpu.VMEM((G, C), jnp.int32),
            pltpu.VMEM((G, C), jnp.int32),
            pltpu.VMEM((G, C), jnp.int32),
            pltpu.VMEM((G, C), jnp.int32),
        ]
        + [pltpu.SemaphoreType.DMA] * 6,
    )
    return f(entx, fused, heads)


_ROWS = 1000


def _pack_body(e_ref, rl_ref, o_ref):
    r = pl.program_id(1)
    x = e_ref[...] * rl_ref[r, :][None, :]
    lo = jnp.concatenate([x[:, 32 * k: 32 * k + 16] for k in range(4)], axis=1)
    hi = jnp.concatenate([x[:, 32 * k + 16: 32 * k + 32] for k in range(4)],
                         axis=1)
    lo_u = jax.lax.bitcast_convert_type(
        lo.astype(jnp.bfloat16), jnp.uint16).astype(jnp.uint32)
    hi_u = jax.lax.bitcast_convert_type(
        hi.astype(jnp.bfloat16), jnp.uint16).astype(jnp.uint32)
    o_ref[...] = jax.lax.bitcast_convert_type((hi_u << 16) | lo_u, jnp.int32)


def _build_entx(ent, rel):
    return pl.pallas_call(
        _pack_body,
        grid=(N_NODES // _ROWS, N_REL),
        in_specs=[
            pl.BlockSpec((_ROWS, D), lambda i, r: (i, 0)),
            pl.BlockSpec((N_REL, D), lambda i, r: (0, 0)),
        ],
        out_specs=pl.BlockSpec(
            (_ROWS, PW), lambda i, r: (r * (N_NODES // _ROWS) + i, 0)),
        out_shape=jax.ShapeDtypeStruct((N_REL * N_NODES, PW), jnp.int32),
    )(ent, rel)


def _comb_body(p_ref, r_ref, n_ref, o_ref):
    s = p_ref[0] + p_ref[1]
    norm = jnp.sqrt(jnp.sum(s * s, axis=1, keepdims=True))
    n = s / jnp.maximum(norm, 1e-12)
    n_ref[...] = n
    o_ref[...] = r_ref[...] + n


def _combine_normalize(parts, res):
    grid = (N_NODES // _ROWS,)
    return pl.pallas_call(
        _comb_body,
        grid=grid,
        in_specs=[
            pl.BlockSpec((NC, _ROWS, D), lambda i: (0, i, 0)),
            pl.BlockSpec((_ROWS, D), lambda i: (i, 0)),
        ],
        out_specs=[
            pl.BlockSpec((_ROWS, D), lambda i: (i, 0)),
            pl.BlockSpec((_ROWS, D), lambda i: (i, 0)),
        ],
        out_shape=[
            jax.ShapeDtypeStruct((N_NODES, D), jnp.float32),
            jax.ShapeDtypeStruct((N_NODES, D), jnp.float32),
        ],
    )(parts, res)


def kernel(ego_embed, edge_index, edge_type, relation_embed, dropout):
    n_edges = edge_index.shape[1]
    epw = n_edges // NW
    pad = EPW_PAD - epw

    def _prep(x, fill):
        x = x.astype(jnp.int32).reshape(NW, epw)
        x = jnp.pad(x, ((0, 0), (0, pad)), constant_values=fill)
        return x.reshape(NW, NGRP, G, C)

    heads = _prep(edge_index[0], TRASH)
    tails = edge_index[1].astype(jnp.int32)
    types = edge_type.astype(jnp.int32)
    fused = _prep(types * N_NODES + tails, 0)

    res = ego_embed
    ego = ego_embed
    for _ in range(N_HOPS):
        entx = _build_entx(ego, relation_embed)
        parts = _sc_aggregate(entx, fused, heads)
        ego, res = _combine_normalize(parts, res)
    return res


# speedup vs baseline: 1.0001x; 1.0001x over previous
import jax
import jax.numpy as jnp
from jax import lax
from jax.experimental import pallas as pl
from jax.experimental.pallas import tpu as pltpu
from jax.experimental.pallas import tpu_sc as plsc

N_NODES = 10000
D = 128
PW = D // 2
N_REL = 16
N_HOPS = 2

NC = 2
NS = 16
NW = NC * NS
C = 64
G = 8
NGRP = 20
QHALF = NGRP // 2
NCHUNK = NGRP * G
EPW_PAD = NCHUNK * C
ACC_ROWS = 10240
TRASH = N_NODES
ZPT = ACC_ROWS // NS


def _agg_body(entx_hbm, fused_hbm, heads_hbm, out_hbm,
              acc, prow0, prow1, rows0, rows1, f0, f1, h0, h1,
              gsem0, gsem1, ssem0, ssem1, isem0, isem1):
    cid = lax.axis_index("c")
    sid = lax.axis_index("s")
    wid = sid * NC + cid

    prow = (prow0, prow1)
    rows = (rows0, rows1)
    fgrp = (f0, f1)
    hgrp = (h0, h1)
    gsem = (gsem0, gsem1)
    ssem = (ssem0, ssem1)
    isem = (isem0, isem1)

    def _zrow(r, carry):
        for c8 in range(8):
            rows0[r, pl.ds(c8 * 16, 16)] = jnp.zeros((16,), jnp.float32)
        return carry
    lax.fori_loop(0, C, _zrow, 0)
    for j in range(ZPT // C):
        pltpu.sync_copy(rows0, acc.at[pl.ds(sid * ZPT + j * C, C)])

    plsc.subcore_barrier()

    def _start_idx(q, p):
        pltpu.async_copy(fused_hbm.at[wid, q], fgrp[p], isem[p])
        pltpu.async_copy(heads_hbm.at[wid, q], hgrp[p], isem[p])

    def _wait_idx(q, p):
        pltpu.make_async_copy(fused_hbm.at[wid, q], fgrp[p], isem[p]).wait()
        pltpu.make_async_copy(heads_hbm.at[wid, q], hgrp[p], isem[p]).wait()

    def _start_gather(idx_ref, b):
        pltpu.async_copy(entx_hbm.at[idx_ref], prow[b], gsem[b])

    def _wait_gather(idx_ref, b):
        pltpu.make_async_copy(entx_hbm.at[idx_ref], prow[b], gsem[b]).wait()

    def _wait_scatter(j, p, b):
        pltpu.make_async_copy(rows[b], acc.at[hgrp[p].at[j]], ssem[b]).wait()

    def _unpack(b):
        src = prow[b]
        dst = rows[b]

        def _row(r, carry):
            for k in range(PW // 16):
                v = src[r, pl.ds(16 * k, 16)]
                a = plsc.bitcast(lax.shift_left(v, 16), jnp.float32)
                bb = plsc.bitcast(
                    lax.bitwise_and(v, jnp.int32(-65536)), jnp.float32)
                dst[r, pl.ds(32 * k, 16)] = a
                dst[r, pl.ds(32 * k + 16, 16)] = bb
            return carry
        lax.fori_loop(0, C, _row, 0)

    _start_idx(0, 0)
    _wait_idx(0, 0)
    _start_gather(fgrp[0].at[0], 0)

    def _step(t, carry):
        for p in range(2):
            q = 2 * t + p
            for j in range(G):
                b = j % 2
                if j == 2:
                    if p == 0:
                        _start_idx(q + 1, 1)
                    else:
                        @pl.when(t < QHALF - 1)
                        def _():
                            _start_idx(q + 1, 0)
                if j < G - 1:
                    _start_gather(fgrp[p].at[j + 1], 1 - b)
                elif p == 0:
                    _wait_idx(q + 1, 1)
                    _start_gather(fgrp[1].at[0], 1 - b)
                else:
                    @pl.when(t < QHALF - 1)
                    def _():
                        _wait_idx(q + 1, 0)
                        _start_gather(fgrp[0].at[0], 1 - b)
                _wait_gather(fgrp[p].at[j], b)
                if p == 0 and j < 2:
                    @pl.when(t > 0)
                    def _():
                        _wait_scatter(G - 2 + j, 1, b)
                elif j < 2:
                    _wait_scatter(G - 2 + j, 0, b)
                else:
                    _wait_scatter(j - 2, p, b)
                _unpack(b)
                pltpu.async_copy(rows[b], acc.at[hgrp[p].at[j]], ssem[b],
                                 add=True)
        return carry

    lax.fori_loop(0, QHALF, _step, 0)
    _wait_scatter(G - 2, 1, 0)
    _wait_scatter(G - 1, 1, 1)

    plsc.subcore_barrier()

    for j in range(ZPT // C):
        base = sid * ZPT + j * C
        pltpu.sync_copy(acc.at[pl.ds(base, C)], rows0)
        pltpu.sync_copy(rows0, out_hbm.at[cid, pl.ds(base, C)])


@jax.jit
def _sc_aggregate(entx, fused, heads):
    mesh = plsc.VectorSubcoreMesh(core_axis_name="c", subcore_axis_name="s")
    f = pl.kernel(
        _agg_body,
        out_type=jax.ShapeDtypeStruct((NC, ACC_ROWS, D), jnp.float32),
        mesh=mesh,
        compiler_params=pltpu.CompilerParams(
            needs_layout_passes=False, use_tc_tiling_on_sc=False
        ),
        scratch_types=[
            pltpu.VMEM_SHARED((ACC_ROWS, D), jnp.float32),
            pltpu.VMEM((C, PW), jnp.int32),
            pltpu.VMEM((C, PW), jnp.int32),
            pltpu.VMEM((C, D), jnp.float32),
            pltpu.VMEM((C, D), jnp.float32),
            pltpu.VMEM((G, C), jnp.int32),
            pltpu.VMEM((G, C), jnp.int32),
            pltpu.VMEM((G, C), jnp.int32),
            pltpu.VMEM((G, C), jnp.int32),
        ]
        + [pltpu.SemaphoreType.DMA] * 6,
    )
    return f(entx, fused, heads)


_ROWS = 1000


def _pack_body(e_ref, rl_ref, o_ref):
    r = pl.program_id(1)
    x = e_ref[...] * rl_ref[r, :][None, :]
    lo = jnp.concatenate([x[:, 32 * k: 32 * k + 16] for k in range(4)], axis=1)
    hi = jnp.concatenate([x[:, 32 * k + 16: 32 * k + 32] for k in range(4)],
                         axis=1)
    lo_u = jax.lax.bitcast_convert_type(
        lo.astype(jnp.bfloat16), jnp.uint16).astype(jnp.uint32)
    hi_u = jax.lax.bitcast_convert_type(
        hi.astype(jnp.bfloat16), jnp.uint16).astype(jnp.uint32)
    o_ref[...] = jax.lax.bitcast_convert_type((hi_u << 16) | lo_u, jnp.int32)


def _build_entx(ent, rel):
    return pl.pallas_call(
        _pack_body,
        grid=(N_NODES // _ROWS, N_REL),
        in_specs=[
            pl.BlockSpec((_ROWS, D), lambda i, r: (i, 0)),
            pl.BlockSpec((N_REL, D), lambda i, r: (0, 0)),
        ],
        out_specs=pl.BlockSpec(
            (_ROWS, PW), lambda i, r: (r * (N_NODES // _ROWS) + i, 0)),
        out_shape=jax.ShapeDtypeStruct((N_REL * N_NODES, PW), jnp.int32),
    )(ent, rel)


def _comb_body(p_ref, r_ref, n_ref, o_ref):
    s = p_ref[0] + p_ref[1]
    norm = jnp.sqrt(jnp.sum(s * s, axis=1, keepdims=True))
    n = s / jnp.maximum(norm, 1e-12)
    n_ref[...] = n
    o_ref[...] = r_ref[...] + n


def _combine_normalize(parts, res):
    grid = (N_NODES // _ROWS,)
    return pl.pallas_call(
        _comb_body,
        grid=grid,
        in_specs=[
            pl.BlockSpec((NC, _ROWS, D), lambda i: (0, i, 0)),
            pl.BlockSpec((_ROWS, D), lambda i: (i, 0)),
        ],
        out_specs=[
            pl.BlockSpec((_ROWS, D), lambda i: (i, 0)),
            pl.BlockSpec((_ROWS, D), lambda i: (i, 0)),
        ],
        out_shape=[
            jax.ShapeDtypeStruct((N_NODES, D), jnp.float32),
            jax.ShapeDtypeStruct((N_NODES, D), jnp.float32),
        ],
    )(parts, res)


def kernel(ego_embed, edge_index, edge_type, relation_embed, dropout):
    n_edges = edge_index.shape[1]
    epw = n_edges // NW
    pad = EPW_PAD - epw

    def _prep(x, fill):
        x = x.astype(jnp.int32).reshape(NW, epw)
        x = jnp.pad(x, ((0, 0), (0, pad)), constant_values=fill)
        return x.reshape(NW, NGRP, G, C)

    heads = _prep(edge_index[0], TRASH)
    tails = edge_index[1].astype(jnp.int32)
    types = edge_type.astype(jnp.int32)
    fused = _prep(types * N_NODES + tails, 0)

    res = ego_embed
    ego = ego_embed
    for _ in range(N_HOPS):
        entx = _build_entx(ego, relation_embed)
        parts = _sc_aggregate(entx, fused, heads)
        ego, res = _combine_normalize(parts, res)
    return res
